# baseline (speedup 1.0000x reference)
import functools

import jax
import jax.numpy as jnp
from jax import lax
from jax.experimental import pallas as pl
from jax.experimental.pallas import tpu as pltpu
from jax.experimental.pallas import tpu_sc as plsc

_B = 16384
_D = 64
_DP = 128
_NC = 2
_NS = 16
_NW = _NC * _NS
_BPW = _B // _NW
_CHUNK = 128
_NCHUNK = _BPW // _CHUNK
_GROUPS = _BPW // 16

_mesh = plsc.VectorSubcoreMesh(core_axis_name="c", subcore_axis_name="s")


@functools.partial(
    pl.kernel,
    mesh=_mesh,
    out_type=[
        jax.ShapeDtypeStruct((_B, _DP), jnp.float32),
        jax.ShapeDtypeStruct((_B,), jnp.float32),
    ],
    scratch_types=[
        pltpu.VMEM((_NCHUNK, _CHUNK), jnp.int32),
        pltpu.VMEM((_BPW, _DP), jnp.float32),
        pltpu.VMEM((_BPW,), jnp.float32),
        pltpu.SemaphoreType.DMA,
        pltpu.SemaphoreType.DMA,
    ],
    compiler_params=pltpu.CompilerParams(
        needs_layout_passes=False, use_tc_tiling_on_sc=True),
)
def _sc_gather_loss(idx_hbm, table_hbm, feat_hbm, loss_hbm,
                    idx_v, rows_v, loss_v, gsem, osem):
    wid = lax.axis_index("s") * _NC + lax.axis_index("c")
    base = wid * _BPW

    for j in range(_NCHUNK):
        pltpu.sync_copy(
            idx_hbm.at[pl.ds(base + j * _CHUNK, _CHUNK)], idx_v.at[j])

    copies = []
    for j in range(_NCHUNK):
        copies.append(pltpu.async_copy(
            table_hbm.at[idx_v.at[j]],
            rows_v.at[pl.ds(j * _CHUNK, _CHUNK)],
            gsem))
    for c in copies:
        c.wait()

    out_copy = pltpu.async_copy(
        rows_v, feat_hbm.at[pl.ds(base, _BPW)], osem)

    lanes = lax.iota(jnp.int32, 16)

    def group_body(g, carry):
        row_ids = g * 16 + lanes
        acc = jnp.zeros((16,), jnp.float32)
        for c in range(_D):
            col = jnp.full((16,), c, jnp.int32)
            v = plsc.load_gather(rows_v, [row_ids, col])
            acc = acc + v * v
        loss_v[pl.ds(g * 16, 16)] = acc * (0.1 / _D)
        return carry

    lax.fori_loop(0, _GROUPS, group_body, 0)

    pltpu.sync_copy(loss_v, loss_hbm.at[pl.ds(base, _BPW)])
    out_copy.wait()


def kernel(input, kernel):
    idx = jnp.asarray(input, jnp.int32)
    table_padded = jnp.pad(kernel, ((0, 0), (0, _DP - _D)))
    feat, loss = _sc_gather_loss(idx, table_padded)
    return feat[:, :_D], loss.reshape(_B, 1)


# speedup vs baseline: 2.6375x; 2.6375x over previous
import functools

import jax
import jax.numpy as jnp
from jax import lax
from jax.experimental import pallas as pl
from jax.experimental.pallas import tpu as pltpu
from jax.experimental.pallas import tpu_sc as plsc

_B = 16384
_D = 64
_DP = 128
_NC = 2
_NS = 16
_NW = _NC * _NS
_KC = 896
_NCHUNKS = 1116
_TMAX = 35
_KEYS = 1000000
_COVERED = _KC * _NCHUNKS
_CAP = 704
_PIECE = 2048
_NPIECE = _B // _PIECE
_RING = 16
_CBYTES = 64 * _KC * 4

_mesh = plsc.VectorSubcoreMesh(core_axis_name="c", subcore_axis_name="s")


@functools.partial(
    pl.kernel,
    mesh=_mesh,
    out_type=jax.ShapeDtypeStruct((_B, _DP), jnp.float32),
    scratch_types=[
        pltpu.VMEM((64, _KC), jnp.float32),
        pltpu.VMEM((64, _KC), jnp.float32),
        pltpu.VMEM((_CAP,), jnp.int32),
        pltpu.VMEM((_CAP,), jnp.int32),
        pltpu.VMEM((_PIECE,), jnp.int32),
        pltpu.VMEM((_RING, _DP), jnp.float32),
        pltpu.SMEM((1,), jnp.int32),
        pltpu.SemaphoreType.DMA,
        pltpu.SemaphoreType.DMA,
        pltpu.SemaphoreType.DMA,
    ],
    compiler_params=pltpu.CompilerParams(
        needs_layout_passes=False, use_tc_tiling_on_sc=True),
)
def _sc_stream_gather(idx_hbm, tablet_hbm, out_hbm,
                      buf_a, buf_b, bkeys, bpos, piece, ring, served_ref,
                      sem_a, sem_b, osem):
    wid = lax.axis_index("s") * _NC + lax.axis_index("c")
    lanes = lax.iota(jnp.int32, 16)
    served_ref[0] = 0

    ones = jnp.full((16,), 1, jnp.int32)
    probe = plsc.cumsum(ones)
    excl_flag = jnp.where(probe[0] == 0, jnp.int32(1), jnp.int32(0))

    def inclusive_cumsum(x):
        return plsc.cumsum(x) + excl_flag * x

    def chunk_of(t):
        return wid + _NW * t

    def issue_chunk(t, buf, sem):
        c = chunk_of(t)

        @pl.when(c < _NCHUNKS)
        def _():
            pltpu.async_copy(
                tablet_hbm.at[:, pl.ds(c * _KC, _KC)], buf, sem)

    issue_chunk(0, buf_a, sem_a)
    issue_chunk(1, buf_b, sem_b)

    for v in range(_CAP // 16):
        bkeys[pl.ds(v * 16, 16)] = jnp.full((16,), jnp.int32(2**30))

    off = jnp.int32(0)
    for p in range(_NPIECE):
        pltpu.sync_copy(idx_hbm.at[pl.ds(p * _PIECE, _PIECE)], piece)

        def scan_body(v, off):
            kv = piece[pl.ds(v * 16, 16)]
            kcl = jnp.minimum(kv, jnp.int32(_COVERED - 1))
            cid = kcl // _KC
            mine = (cid % _NW) == wid
            posv = p * _PIECE + v * 16 + lanes
            inc = inclusive_cumsum(mine.astype(jnp.int32))
            dest = off + inc - 1
            plsc.store_scatter(bkeys, [dest], kcl, mask=mine)
            plsc.store_scatter(bpos, [dest], posv, mask=mine)
            return off + inc[15]

        off = lax.fori_loop(0, _PIECE // 16, scan_body, off)

    def serve_key(m1, kv, pv, lo, buf):
        key = jnp.sum(jnp.where(m1, kv, 0))
        pos = jnp.sum(jnp.where(m1, pv, 0))
        offl = key - lo
        col = jnp.full((16,), 0, jnp.int32) + offl
        srv = served_ref[0]
        r = lax.rem(srv, jnp.int32(_RING))

        @pl.when(jnp.logical_and(r == 0, srv > 0))
        def _():
            pltpu.make_async_copy(
                out_hbm.at[pl.ds(0, _RING)], ring, osem).wait()

        acc = jnp.zeros((16,), jnp.float32)
        for d16 in range(4):
            rows = d16 * 16 + lanes
            g = plsc.load_gather(buf, [rows, col])
            ring[r, pl.ds(d16 * 16, 16)] = g
            acc = acc + g * g
        loss = jnp.sum(acc) * (0.1 / _D)
        ring[r, pl.ds(_D, 16)] = jnp.zeros((16,), jnp.float32) + loss
        pltpu.async_copy(
            ring.at[pl.ds(r, 1)], out_hbm.at[pl.ds(pos, 1)], osem)
        served_ref[0] = srv + 1

    def serve_chunk(t, buf):
        c = chunk_of(t)
        lo = c * _KC

        def vreg_body(v, carry):
            kv = bkeys[pl.ds(v * 16, 16)]
            pv = bpos[pl.ds(v * 16, 16)]
            m = jnp.logical_and(kv >= lo, kv < lo + _KC)
            cnt = plsc.all_reduce_population_count(m)[0]

            def body(i, m):
                sel = inclusive_cumsum(m.astype(jnp.int32))
                m1 = jnp.logical_and(m, sel == 1)
                serve_key(m1, kv, pv, lo, buf)
                return jnp.logical_and(m, jnp.logical_not(m1))

            lax.fori_loop(0, cnt, body, m)
            return carry

        lax.fori_loop(0, _CAP // 16, vreg_body, 0)

    def main_body(t, carry):
        c = chunk_of(t)

        @pl.when(c < _NCHUNKS)
        def _():
            @pl.when(t % 2 == 0)
            def _():
                pltpu.make_async_copy(
                    tablet_hbm.at[:, pl.ds(0, _KC)], buf_a, sem_a).wait()
                serve_chunk(t, buf_a)
                issue_chunk(t + 2, buf_a, sem_a)

            @pl.when(t % 2 == 1)
            def _():
                pltpu.make_async_copy(
                    tablet_hbm.at[:, pl.ds(0, _KC)], buf_b, sem_b).wait()
                serve_chunk(t, buf_b)
                issue_chunk(t + 2, buf_b, sem_b)

        return carry

    lax.fori_loop(0, _TMAX, main_body, 0)

    srv = served_ref[0]
    rem = jnp.where(
        srv > 0, srv - _RING * ((srv - 1) // _RING), jnp.int32(0))

    def drain_body(i, carry):
        pltpu.make_async_copy(
            out_hbm.at[pl.ds(0, 1)], ring.at[pl.ds(0, 1)], osem).wait()
        return carry

    lax.fori_loop(0, rem, drain_body, 0)


def kernel(input, kernel):
    idx = jnp.asarray(input, jnp.int32)
    out = _sc_stream_gather(idx, kernel.T)
    feat = out[:, :_D]
    loss = out[:, _D:_D + 1]
    tail = kernel[_COVERED:]
    is_tail = idx >= _COVERED
    local = jnp.clip(idx - _COVERED, 0, _KEYS - _COVERED - 1)
    onehot = jnp.where(
        is_tail[:, None],
        (local[:, None] == jnp.arange(_KEYS - _COVERED)[None, :]).astype(
            jnp.float32),
        0.0)
    feat_tail = jnp.matmul(onehot, tail, precision="highest")
    feat = jnp.where(is_tail[:, None], feat_tail, feat)
    loss_tail = 0.1 * jnp.mean(
        jnp.square(feat_tail), axis=1, keepdims=True)
    loss = jnp.where(is_tail[:, None], loss_tail, loss)
    return feat, loss


# speedup vs baseline: 2.8513x; 1.0810x over previous
import functools

import jax
import jax.numpy as jnp
from jax import lax
from jax.experimental import pallas as pl
from jax.experimental.pallas import tpu as pltpu
from jax.experimental.pallas import tpu_sc as plsc

_B = 16384
_D = 64
_DP = 128
_NC = 2
_NS = 16
_NW = _NC * _NS
_KC = 896
_NCHUNKS = 1116
_TMAX = 35
_KEYS = 1000000
_COVERED = _KC * _NCHUNKS
_CAP = 704
_PIECE = 2048
_NPIECE = _B // _PIECE
_RING = 16
_CBYTES = 64 * _KC * 4

_mesh = plsc.VectorSubcoreMesh(core_axis_name="c", subcore_axis_name="s")


@functools.partial(
    pl.kernel,
    mesh=_mesh,
    out_type=jax.ShapeDtypeStruct((_B, _DP), jnp.float32),
    scratch_types=[
        pltpu.VMEM((64, _KC), jnp.float32),
        pltpu.VMEM((64, _KC), jnp.float32),
        pltpu.VMEM((64, 128), jnp.float32),
        pltpu.VMEM((_CAP,), jnp.int32),
        pltpu.VMEM((_CAP,), jnp.int32),
        pltpu.VMEM((_PIECE,), jnp.int32),
        pltpu.VMEM((_RING, _DP), jnp.float32),
        pltpu.SMEM((1,), jnp.int32),
        pltpu.SemaphoreType.DMA,
        pltpu.SemaphoreType.DMA,
        pltpu.SemaphoreType.DMA,
    ],
    compiler_params=pltpu.CompilerParams(
        needs_layout_passes=False, use_tc_tiling_on_sc=True),
)
def _sc_stream_gather(idx_hbm, tablet_hbm, tail_hbm, out_hbm,
                      buf_a, buf_b, tail_v, bkeys, bpos, piece, ring,
                      served_ref, sem_a, sem_b, osem):
    wid = lax.axis_index("s") * _NC + lax.axis_index("c")
    lanes = lax.iota(jnp.int32, 16)
    served_ref[0] = 0

    ones = jnp.full((16,), 1, jnp.int32)
    probe = plsc.cumsum(ones)
    excl_flag = jnp.where(probe[0] == 0, jnp.int32(1), jnp.int32(0))

    def inclusive_cumsum(x):
        return plsc.cumsum(x) + excl_flag * x

    def chunk_of(t):
        return wid + _NW * t

    def issue_chunk(t, buf, sem):
        c = chunk_of(t)

        @pl.when(c < _NCHUNKS)
        def _():
            pltpu.async_copy(
                tablet_hbm.at[:, pl.ds(c * _KC, _KC)], buf, sem)

    issue_chunk(0, buf_a, sem_a)
    issue_chunk(1, buf_b, sem_b)
    pltpu.sync_copy(tail_hbm, tail_v)

    for v in range(_CAP // 16):
        bkeys[pl.ds(v * 16, 16)] = jnp.full((16,), jnp.int32(2**30))

    off = jnp.int32(0)
    for p in range(_NPIECE):
        pltpu.sync_copy(idx_hbm.at[pl.ds(p * _PIECE, _PIECE)], piece)

        def scan_body(v, off):
            kv = piece[pl.ds(v * 16, 16)]
            cid = kv // _KC
            mine = (cid % _NW) == wid
            posv = p * _PIECE + v * 16 + lanes
            inc = inclusive_cumsum(mine.astype(jnp.int32))
            dest = off + inc - 1
            plsc.store_scatter(bkeys, [dest], kv, mask=mine)
            plsc.store_scatter(bpos, [dest], posv, mask=mine)
            return off + inc[15]

        off = lax.fori_loop(0, _PIECE // 16, scan_body, off)

    def serve_key(m1, kv, pv, lo, buf):
        key = jnp.sum(jnp.where(m1, kv, 0))
        pos = jnp.sum(jnp.where(m1, pv, 0))
        offl = key - lo
        col = jnp.full((16,), 0, jnp.int32) + offl
        srv = served_ref[0]
        r = lax.rem(srv, jnp.int32(_RING))

        @pl.when(jnp.logical_and(r == 0, srv > 0))
        def _():
            pltpu.make_async_copy(
                out_hbm.at[pl.ds(0, _RING)], ring, osem).wait()

        acc = jnp.zeros((16,), jnp.float32)
        for d16 in range(4):
            rows = d16 * 16 + lanes
            g = plsc.load_gather(buf, [rows, col])
            ring[r, pl.ds(d16 * 16, 16)] = g
            acc = acc + g * g
        loss = jnp.sum(acc) * (0.1 / _D)
        ring[r, pl.ds(_D, 16)] = jnp.zeros((16,), jnp.float32) + loss
        pltpu.async_copy(
            ring.at[pl.ds(r, 1)], out_hbm.at[pl.ds(pos, 1)], osem)
        served_ref[0] = srv + 1

    def serve_chunk(t, buf):
        c = chunk_of(t)
        lo = c * _KC

        def vreg_body(v, carry):
            kv = bkeys[pl.ds(v * 16, 16)]
            pv = bpos[pl.ds(v * 16, 16)]
            m = jnp.logical_and(kv >= lo, kv < lo + _KC)
            cnt = plsc.all_reduce_population_count(m)[0]

            def body(i, m):
                sel = inclusive_cumsum(m.astype(jnp.int32))
                m1 = jnp.logical_and(m, sel == 1)
                serve_key(m1, kv, pv, lo, buf)
                return jnp.logical_and(m, jnp.logical_not(m1))

            lax.fori_loop(0, cnt, body, m)
            return carry

        lax.fori_loop(0, _CAP // 16, vreg_body, 0)

    def main_body(t, carry):
        c = chunk_of(t)

        @pl.when(c == _NCHUNKS)
        def _():
            serve_chunk(t, tail_v)

        @pl.when(c < _NCHUNKS)
        def _():
            @pl.when(t % 2 == 0)
            def _():
                pltpu.make_async_copy(
                    tablet_hbm.at[:, pl.ds(0, _KC)], buf_a, sem_a).wait()
                serve_chunk(t, buf_a)
                issue_chunk(t + 2, buf_a, sem_a)

            @pl.when(t % 2 == 1)
            def _():
                pltpu.make_async_copy(
                    tablet_hbm.at[:, pl.ds(0, _KC)], buf_b, sem_b).wait()
                serve_chunk(t, buf_b)
                issue_chunk(t + 2, buf_b, sem_b)

        return carry

    lax.fori_loop(0, _TMAX, main_body, 0)

    srv = served_ref[0]
    rem = jnp.where(
        srv > 0, srv - _RING * ((srv - 1) // _RING), jnp.int32(0))

    def drain_body(i, carry):
        pltpu.make_async_copy(
            out_hbm.at[pl.ds(0, 1)], ring.at[pl.ds(0, 1)], osem).wait()
        return carry

    lax.fori_loop(0, rem, drain_body, 0)


def kernel(input, kernel):
    idx = jnp.asarray(input, jnp.int32)
    tail = jnp.pad(kernel[_COVERED:].T, ((0, 0), (0, 128 - (_KEYS - _COVERED))))
    out = _sc_stream_gather(idx, kernel.T, tail)
    feat = out[:, :_D]
    loss = out[:, _D:_D + 1]
    return feat, loss


# speedup vs baseline: 3.1098x; 1.0907x over previous
import functools

import jax
import jax.numpy as jnp
from jax import lax
from jax.experimental import pallas as pl
from jax.experimental.pallas import tpu as pltpu
from jax.experimental.pallas import tpu_sc as plsc

_B = 16384
_D = 64
_DP = 128
_NC = 2
_NS = 16
_NW = _NC * _NS
_KC = 512
_NCHUNKS = 1953
_TMAX = 62
_KEYS = 1000000
_COVERED = _KC * _NCHUNKS
_CAP = 704
_PIECE = 2048
_NPIECE = _B // _PIECE
_RING = 16
_CBYTES = 64 * _KC * 4

_mesh = plsc.VectorSubcoreMesh(core_axis_name="c", subcore_axis_name="s")


@functools.partial(
    pl.kernel,
    mesh=_mesh,
    out_type=jax.ShapeDtypeStruct((_B, _DP), jnp.float32),
    scratch_types=[
        pltpu.VMEM((64, _KC), jnp.float32),
        pltpu.VMEM((64, _KC), jnp.float32),
        pltpu.VMEM((64, _KC), jnp.float32),
        pltpu.VMEM((64, 128), jnp.float32),
        pltpu.VMEM((_CAP,), jnp.int32),
        pltpu.VMEM((_CAP,), jnp.int32),
        pltpu.VMEM((_PIECE,), jnp.int32),
        pltpu.VMEM((_RING, _DP), jnp.float32),
        pltpu.SMEM((1,), jnp.int32),
        pltpu.SemaphoreType.DMA,
        pltpu.SemaphoreType.DMA,
        pltpu.SemaphoreType.DMA,
        pltpu.SemaphoreType.DMA,
    ],
    compiler_params=pltpu.CompilerParams(
        needs_layout_passes=False, use_tc_tiling_on_sc=True),
)
def _sc_stream_gather(idx_hbm, tablet_hbm, tail_hbm, out_hbm,
                      buf_a, buf_b, buf_c, tail_v, bkeys, bpos, piece, ring,
                      served_ref, sem_a, sem_b, sem_c, osem):
    wid = lax.axis_index("s") * _NC + lax.axis_index("c")
    lanes = lax.iota(jnp.int32, 16)
    served_ref[0] = 0

    ones = jnp.full((16,), 1, jnp.int32)
    probe = plsc.cumsum(ones)
    excl_flag = jnp.where(probe[0] == 0, jnp.int32(1), jnp.int32(0))

    def inclusive_cumsum(x):
        return plsc.cumsum(x) + excl_flag * x

    def chunk_of(t):
        return wid + _NW * t

    def issue_chunk(t, buf, sem):
        c = chunk_of(t)

        @pl.when(c < _NCHUNKS)
        def _():
            pltpu.async_copy(
                tablet_hbm.at[:, pl.ds(c * _KC, _KC)], buf, sem)

    issue_chunk(0, buf_a, sem_a)
    issue_chunk(1, buf_b, sem_b)
    issue_chunk(2, buf_c, sem_c)
    pltpu.sync_copy(tail_hbm, tail_v)

    for v in range(_CAP // 16):
        bkeys[pl.ds(v * 16, 16)] = jnp.full((16,), jnp.int32(2**30))

    off = jnp.int32(0)
    for p in range(_NPIECE):
        pltpu.sync_copy(idx_hbm.at[pl.ds(p * _PIECE, _PIECE)], piece)

        def scan_body(v, off):
            kv = piece[pl.ds(v * 16, 16)]
            cid = kv // _KC
            mine = (cid % _NW) == wid
            posv = p * _PIECE + v * 16 + lanes
            inc = inclusive_cumsum(mine.astype(jnp.int32))
            dest = off + inc - 1
            plsc.store_scatter(bkeys, [dest], kv, mask=mine)
            plsc.store_scatter(bpos, [dest], posv, mask=mine)
            return off + inc[15]

        off = lax.fori_loop(0, _PIECE // 16, scan_body, off)

    def serve_key(m1, kv, pv, lo, buf):
        key = jnp.sum(jnp.where(m1, kv, 0))
        pos = jnp.sum(jnp.where(m1, pv, 0))
        offl = key - lo
        col = jnp.full((16,), 0, jnp.int32) + offl
        srv = served_ref[0]
        r = lax.rem(srv, jnp.int32(_RING))

        @pl.when(jnp.logical_and(r == 0, srv > 0))
        def _():
            pltpu.make_async_copy(
                out_hbm.at[pl.ds(0, _RING)], ring, osem).wait()

        acc = jnp.zeros((16,), jnp.float32)
        for d16 in range(4):
            rows = d16 * 16 + lanes
            g = plsc.load_gather(buf, [rows, col])
            ring[r, pl.ds(d16 * 16, 16)] = g
            acc = acc + g * g
        loss = jnp.sum(acc) * (0.1 / _D)
        ring[r, pl.ds(_D, 16)] = jnp.zeros((16,), jnp.float32) + loss
        pltpu.async_copy(
            ring.at[pl.ds(r, 1)], out_hbm.at[pl.ds(pos, 1)], osem)
        served_ref[0] = srv + 1

    def serve_chunk(t, buf):
        c = chunk_of(t)
        lo = c * _KC

        def vreg_body(v, carry):
            kv = bkeys[pl.ds(v * 16, 16)]
            pv = bpos[pl.ds(v * 16, 16)]
            m = jnp.logical_and(kv >= lo, kv < lo + _KC)
            cnt = plsc.all_reduce_population_count(m)[0]

            def body(i, m):
                sel = inclusive_cumsum(m.astype(jnp.int32))
                m1 = jnp.logical_and(m, sel == 1)
                serve_key(m1, kv, pv, lo, buf)
                return jnp.logical_and(m, jnp.logical_not(m1))

            lax.fori_loop(0, cnt, body, m)
            return carry

        lax.fori_loop(0, _CAP // 16, vreg_body, 0)

    def main_body(t, carry):
        c = chunk_of(t)

        @pl.when(c == _NCHUNKS)
        def _():
            serve_chunk(t, tail_v)

        @pl.when(c < _NCHUNKS)
        def _():
            @pl.when(t % 3 == 0)
            def _():
                pltpu.make_async_copy(
                    tablet_hbm.at[:, pl.ds(0, _KC)], buf_a, sem_a).wait()
                serve_chunk(t, buf_a)
                issue_chunk(t + 3, buf_a, sem_a)

            @pl.when(t % 3 == 1)
            def _():
                pltpu.make_async_copy(
                    tablet_hbm.at[:, pl.ds(0, _KC)], buf_b, sem_b).wait()
                serve_chunk(t, buf_b)
                issue_chunk(t + 3, buf_b, sem_b)

            @pl.when(t % 3 == 2)
            def _():
                pltpu.make_async_copy(
                    tablet_hbm.at[:, pl.ds(0, _KC)], buf_c, sem_c).wait()
                serve_chunk(t, buf_c)
                issue_chunk(t + 3, buf_c, sem_c)

        return carry

    lax.fori_loop(0, _TMAX, main_body, 0)

    srv = served_ref[0]
    rem = jnp.where(
        srv > 0, srv - _RING * ((srv - 1) // _RING), jnp.int32(0))

    def drain_body(i, carry):
        pltpu.make_async_copy(
            out_hbm.at[pl.ds(0, 1)], ring.at[pl.ds(0, 1)], osem).wait()
        return carry

    lax.fori_loop(0, rem, drain_body, 0)


def kernel(input, kernel):
    idx = jnp.asarray(input, jnp.int32)
    tail = jnp.pad(kernel[_COVERED:].T, ((0, 0), (0, 128 - (_KEYS - _COVERED))))
    out = _sc_stream_gather(idx, kernel.T, tail)
    feat = out[:, :_D]
    loss = out[:, _D:_D + 1]
    return feat, loss
